# baseline (speedup 1.0000x reference)
import functools

import jax
import jax.numpy as jnp
from jax import lax
from jax.experimental import pallas as pl
from jax.experimental.pallas import tpu as pltpu
from jax.experimental.pallas import tpu_sc as plsc

B = 16
N = 4096
D = 64
TEMPERATURE = 0.05

NW = 32
CHUNK = 128
NCHUNK = N // CHUNK
GROUPS = CHUNK // 16


def _bce_weighted(dots, t, wv):
    l = dots * (1.0 / TEMPERATURE)
    al = jnp.abs(l)
    u = jnp.exp(-al)
    z = u / (u + 2.0)
    z2 = z * z
    lp = (2.0 * z) * (1.0 + z2 * (1.0 / 3 + z2 * (1.0 / 5 + z2 * (1.0 / 7 + z2 * (1.0 / 9)))))
    loss = jnp.maximum(l, 0.0) - l * t + lp
    return loss * wv


def _sc_loss_parts(lengths, output_embeddings, supervision_embeddings,
                   supervision_weights, supervision_ratings):
    mesh = plsc.VectorSubcoreMesh(core_axis_name="c", subcore_axis_name="s")

    @functools.partial(
        pl.kernel,
        mesh=mesh,
        compiler_params=pltpu.CompilerParams(needs_layout_passes=False),
        out_type=[
            jax.ShapeDtypeStruct((NW, 16), jnp.float32),
            jax.ShapeDtypeStruct((NW, 16), jnp.float32),
        ],
        scratch_types=[
            pltpu.VMEM((B,), jnp.int32),
            pltpu.VMEM((CHUNK * D,), jnp.float32),
            pltpu.VMEM((CHUNK * D,), jnp.float32),
            pltpu.VMEM((CHUNK,), jnp.float32),
            pltpu.VMEM((CHUNK,), jnp.float32),
            pltpu.VMEM((16,), jnp.float32),
            pltpu.VMEM((16,), jnp.float32),
        ],
    )
    def sc_kernel(len_hbm, oe_hbm, se_hbm, sw_hbm, sr_hbm,
                  wl_out, w_out,
                  len_v, a_v, c_v, w_v, r_v, awl_v, aw_v):
        wid = lax.axis_index("s") * 2 + lax.axis_index("c")
        pltpu.sync_copy(len_hbm, len_v)
        awl_v[...] = jnp.zeros((16,), jnp.float32)
        aw_v[...] = jnp.zeros((16,), jnp.float32)
        lane = lax.iota(jnp.int32, 16)
        len_all = len_v[...]

        for b in range(B):
            j = (wid + 2 * b) & (NCHUNK - 1)
            base = j * CHUNK
            len_b = len_all[b]

            @pl.when(base < len_b)
            def _():
                pltpu.sync_copy(oe_hbm.at[b, pl.ds(base * D, CHUNK * D)], a_v)
                pltpu.sync_copy(se_hbm.at[b, pl.ds(base * D, CHUNK * D)], c_v)
                pltpu.sync_copy(sw_hbm.at[b, pl.ds(base, CHUNK)], w_v)
                pltpu.sync_copy(sr_hbm.at[b, pl.ds(base, CHUNK)], r_v)

                def group_body(g, _):
                    row0 = g * 16
                    rows = row0 + lane
                    rowbase = rows * D

                    def d_body(i, acc):
                        d0 = i * 4
                        for k in range(4):
                            idx = rowbase + (d0 + k)
                            a = plsc.load_gather(a_v, [idx])
                            c = plsc.load_gather(c_v, [idx])
                            acc = acc + a * c
                        return acc

                    dots = lax.fori_loop(0, D // 4, d_body,
                                         jnp.zeros((16,), jnp.float32))
                    t = r_v[pl.ds(row0, 16)]
                    wv = w_v[pl.ds(row0, 16)]
                    valid = (base + rows) < len_b
                    wv = jnp.where(valid, wv, 0.0)
                    awl_v[...] += _bce_weighted(dots, t, wv)
                    aw_v[...] += wv
                    return _

                lax.fori_loop(0, GROUPS, group_body, None)

        pltpu.sync_copy(awl_v, wl_out.at[wid])
        pltpu.sync_copy(aw_v, w_out.at[wid])

    return sc_kernel(lengths,
                     output_embeddings.reshape(B, N * D),
                     supervision_embeddings.reshape(B, N * D),
                     supervision_weights, supervision_ratings)


def kernel(lengths, output_embeddings, supervision_ids, supervision_embeddings,
           supervision_weights, supervision_ratings):
    del supervision_ids
    wl, w = _sc_loss_parts(lengths, output_embeddings, supervision_embeddings,
                           supervision_weights, supervision_ratings)
    return jnp.sum(wl) / jnp.sum(w)


# speedup vs baseline: 1.1205x; 1.1205x over previous
import functools

import jax
import jax.numpy as jnp
from jax import lax
from jax.experimental import pallas as pl
from jax.experimental.pallas import tpu as pltpu
from jax.experimental.pallas import tpu_sc as plsc

B = 16
N = 4096
D = 64
TEMPERATURE = 0.05

NW = 32
CHUNK = 128
NCHUNK = N // CHUNK
GROUPS = CHUNK // 16


def _bce_weighted(dots, t, wv):
    l = dots * (1.0 / TEMPERATURE)
    al = jnp.abs(l)
    u = jnp.exp(-al)
    z = u / (u + 2.0)
    z2 = z * z
    lp = (2.0 * z) * (1.0 + z2 * (1.0 / 3 + z2 * (1.0 / 5 + z2 * (1.0 / 7 + z2 * (1.0 / 9)))))
    loss = jnp.maximum(l, 0.0) - l * t + lp
    return loss * wv


def _sc_loss_parts(lengths, output_embeddings, supervision_embeddings,
                   supervision_weights, supervision_ratings):
    mesh = plsc.VectorSubcoreMesh(core_axis_name="c", subcore_axis_name="s")

    @functools.partial(
        pl.kernel,
        mesh=mesh,
        compiler_params=pltpu.CompilerParams(needs_layout_passes=False),
        out_type=[
            jax.ShapeDtypeStruct((NW, 16), jnp.float32),
            jax.ShapeDtypeStruct((NW, 16), jnp.float32),
        ],
        scratch_types=[
            pltpu.VMEM((B,), jnp.int32),
            pltpu.VMEM((CHUNK, D), jnp.float32),
            pltpu.VMEM((CHUNK, D), jnp.float32),
            pltpu.VMEM((CHUNK,), jnp.float32),
            pltpu.VMEM((CHUNK,), jnp.float32),
            pltpu.VMEM((16,), jnp.float32),
            pltpu.VMEM((16,), jnp.float32),
        ],
    )
    def sc_kernel(len_hbm, oe_hbm, se_hbm, sw_hbm, sr_hbm,
                  wl_out, w_out,
                  len_v, a_v, c_v, w_v, r_v, awl_v, aw_v):
        wid = lax.axis_index("s") * 2 + lax.axis_index("c")
        pltpu.sync_copy(len_hbm, len_v)
        awl_v[...] = jnp.zeros((16,), jnp.float32)
        aw_v[...] = jnp.zeros((16,), jnp.float32)
        lane = lax.iota(jnp.int32, 16)
        len_all = len_v[...]

        for b in range(B):
            j = (wid + 2 * b) & (NCHUNK - 1)
            base = j * CHUNK
            len_b = len_all[b]

            @pl.when(base < len_b)
            def _():
                pltpu.sync_copy(oe_hbm.at[b, pl.ds(base, CHUNK)], a_v)
                pltpu.sync_copy(se_hbm.at[b, pl.ds(base, CHUNK)], c_v)
                pltpu.sync_copy(sw_hbm.at[b, pl.ds(base, CHUNK)], w_v)
                pltpu.sync_copy(sr_hbm.at[b, pl.ds(base, CHUNK)], r_v)

                def group_body(g, _):
                    row0 = g * 16
                    rows = row0 + lane

                    def d_body(i, acc):
                        d0 = i * 4
                        for k in range(4):
                            dvec = jnp.full((16,), d0 + k, jnp.int32)
                            a = plsc.load_gather(a_v, [rows, dvec])
                            c = plsc.load_gather(c_v, [rows, dvec])
                            acc = acc + a * c
                        return acc

                    dots = lax.fori_loop(0, D // 4, d_body,
                                         jnp.zeros((16,), jnp.float32))
                    t = r_v[pl.ds(row0, 16)]
                    wv = w_v[pl.ds(row0, 16)]
                    valid = (base + rows) < len_b
                    wv = jnp.where(valid, wv, 0.0)
                    awl_v[...] += _bce_weighted(dots, t, wv)
                    aw_v[...] += wv
                    return _

                lax.fori_loop(0, GROUPS, group_body, None)

        pltpu.sync_copy(awl_v, wl_out.at[wid])
        pltpu.sync_copy(aw_v, w_out.at[wid])

    return sc_kernel(lengths, output_embeddings, supervision_embeddings,
                     supervision_weights, supervision_ratings)


def kernel(lengths, output_embeddings, supervision_ids, supervision_embeddings,
           supervision_weights, supervision_ratings):
    del supervision_ids
    wl, w = _sc_loss_parts(lengths, output_embeddings, supervision_embeddings,
                           supervision_weights, supervision_ratings)
    return jnp.sum(wl) / jnp.sum(w)


# speedup vs baseline: 1.4345x; 1.2803x over previous
import functools

import jax
import jax.numpy as jnp
from jax import lax
from jax.experimental import pallas as pl
from jax.experimental.pallas import tpu as pltpu
from jax.experimental.pallas import tpu_sc as plsc

B = 16
N = 4096
D = 64
TEMPERATURE = 0.05

NW = 32
CHUNK = 128
NCHUNK = N // CHUNK
GROUPS = CHUNK // 16


def _bce_weighted(dots, t, wv):
    l = dots * (1.0 / TEMPERATURE)
    al = jnp.abs(l)
    u = jnp.exp(-al)
    z = u / (u + 2.0)
    z2 = z * z
    lp = (2.0 * z) * (1.0 + z2 * (1.0 / 3 + z2 * (1.0 / 5 + z2 * (1.0 / 7 + z2 * (1.0 / 9)))))
    loss = jnp.maximum(l, 0.0) - l * t + lp
    return loss * wv


def _sc_loss_parts(lengths, output_embeddings, supervision_embeddings,
                   supervision_weights, supervision_ratings):
    mesh = plsc.VectorSubcoreMesh(core_axis_name="c", subcore_axis_name="s")

    @functools.partial(
        pl.kernel,
        mesh=mesh,
        compiler_params=pltpu.CompilerParams(needs_layout_passes=False),
        out_type=[
            jax.ShapeDtypeStruct((NW, 16), jnp.float32),
            jax.ShapeDtypeStruct((NW, 16), jnp.float32),
        ],
        scratch_types=[
            pltpu.VMEM((B,), jnp.int32),
            pltpu.VMEM((CHUNK, D), jnp.float32),
            pltpu.VMEM((CHUNK, D), jnp.float32),
            pltpu.VMEM((CHUNK, D), jnp.float32),
            pltpu.VMEM((CHUNK, D), jnp.float32),
            pltpu.VMEM((B * CHUNK,), jnp.float32),
            pltpu.VMEM((B * CHUNK,), jnp.float32),
            pltpu.VMEM((16,), jnp.float32),
            pltpu.VMEM((16,), jnp.float32),
            pltpu.SemaphoreType.DMA,
            pltpu.SemaphoreType.DMA,
            pltpu.SemaphoreType.DMA,
        ],
    )
    def sc_kernel(len_hbm, oe_hbm, se_hbm, sw_hbm, sr_hbm,
                  wl_out, w_out,
                  len_v, a0_v, a1_v, c0_v, c1_v, w_v, r_v, awl_v, aw_v,
                  sem0, sem1, semwr):
        wid = lax.axis_index("s") * 2 + lax.axis_index("c")
        lane = lax.iota(jnp.int32, 16)

        for b in range(B):
            base = ((wid + 2 * b) & (NCHUNK - 1)) * CHUNK
            pltpu.async_copy(sw_hbm.at[b, pl.ds(base, CHUNK)],
                             w_v.at[pl.ds(b * CHUNK, CHUNK)], semwr)
            pltpu.async_copy(sr_hbm.at[b, pl.ds(base, CHUNK)],
                             r_v.at[pl.ds(b * CHUNK, CHUNK)], semwr)
        pltpu.sync_copy(len_hbm, len_v)
        for b in range(B):
            base = ((wid + 2 * b) & (NCHUNK - 1)) * CHUNK
            pltpu.make_async_copy(sw_hbm.at[b, pl.ds(base, CHUNK)],
                                  w_v.at[pl.ds(b * CHUNK, CHUNK)], semwr).wait()
            pltpu.make_async_copy(sr_hbm.at[b, pl.ds(base, CHUNK)],
                                  r_v.at[pl.ds(b * CHUNK, CHUNK)], semwr).wait()

        awl_v[...] = jnp.zeros((16,), jnp.float32)
        aw_v[...] = jnp.zeros((16,), jnp.float32)
        len_all = len_v[...]

        def binfo(b):
            base = ((wid + 2 * b) & (NCHUNK - 1)) * CHUNK
            len_b = jnp.max(jnp.where(lane == b, len_all, 0))
            return b * N + base, base, len_b

        def issue(b, a_buf, c_buf, sem):
            grow, base, len_b = binfo(b)

            @pl.when(base < len_b)
            def _():
                pltpu.async_copy(oe_hbm.at[pl.ds(grow, CHUNK)], a_buf, sem)
                pltpu.async_copy(se_hbm.at[pl.ds(grow, CHUNK)], c_buf, sem)

        def compute(b, a_buf, c_buf, sem):
            grow, base, len_b = binfo(b)

            @pl.when(base < len_b)
            def _():
                pltpu.make_async_copy(oe_hbm.at[pl.ds(grow, CHUNK)], a_buf, sem).wait()
                pltpu.make_async_copy(se_hbm.at[pl.ds(grow, CHUNK)], c_buf, sem).wait()

                def group_body(g, _):
                    row0 = g * 16
                    rows = row0 + lane
                    accs = [jnp.zeros((16,), jnp.float32) for _ in range(4)]
                    for d in range(D):
                        dvec = jnp.full((16,), d, jnp.int32)
                        a = plsc.load_gather(a_buf, [rows, dvec])
                        c = plsc.load_gather(c_buf, [rows, dvec])
                        accs[d & 3] = accs[d & 3] + a * c
                    dots = (accs[0] + accs[1]) + (accs[2] + accs[3])
                    off = b * CHUNK + row0
                    t = r_v[pl.ds(off, 16)]
                    wv = w_v[pl.ds(off, 16)]
                    valid = (base + rows) < len_b
                    wv = jnp.where(valid, wv, 0.0)
                    awl_v[...] += _bce_weighted(dots, t, wv)
                    aw_v[...] += wv
                    return _

                lax.fori_loop(0, GROUPS, group_body, None)

        issue(0, a0_v, c0_v, sem0)

        def pipe_body(i, _):
            b0 = 2 * i
            issue(b0 + 1, a1_v, c1_v, sem1)
            compute(b0, a0_v, c0_v, sem0)

            @pl.when(i < B // 2 - 1)
            def _():
                issue(b0 + 2, a0_v, c0_v, sem0)

            compute(b0 + 1, a1_v, c1_v, sem1)
            return _

        lax.fori_loop(0, B // 2, pipe_body, None)

        pltpu.sync_copy(awl_v, wl_out.at[wid])
        pltpu.sync_copy(aw_v, w_out.at[wid])

    return sc_kernel(lengths,
                     output_embeddings.reshape(B * N, D),
                     supervision_embeddings.reshape(B * N, D),
                     supervision_weights, supervision_ratings)


def kernel(lengths, output_embeddings, supervision_ids, supervision_embeddings,
           supervision_weights, supervision_ratings):
    del supervision_ids
    wl, w = _sc_loss_parts(lengths, output_embeddings, supervision_embeddings,
                           supervision_weights, supervision_ratings)
    return jnp.sum(wl) / jnp.sum(w)


# speedup vs baseline: 2.3276x; 1.6226x over previous
import functools

import jax
import jax.numpy as jnp
from jax import lax
from jax.experimental import pallas as pl
from jax.experimental.pallas import tpu as pltpu
from jax.experimental.pallas import tpu_sc as plsc

B = 16
N = 4096
D = 64
TEMPERATURE = 0.05

NW = 32
CHUNK = 128
NCHUNK = N // CHUNK
GROUPS = CHUNK // 16


def _bce_weighted(dots, t, wv):
    l = dots * (1.0 / TEMPERATURE)
    al = jnp.abs(l)
    u = jnp.exp(-al)
    z = u / (u + 2.0)
    z2 = z * z
    lp = (2.0 * z) * (1.0 + z2 * (1.0 / 3 + z2 * (1.0 / 5 + z2 * (1.0 / 7 + z2 * (1.0 / 9)))))
    loss = jnp.maximum(l, 0.0) - l * t + lp
    return loss * wv


def _sc_loss_parts(lengths, output_embeddings, supervision_embeddings,
                   supervision_weights, supervision_ratings):
    mesh = plsc.VectorSubcoreMesh(core_axis_name="c", subcore_axis_name="s")

    @functools.partial(
        pl.kernel,
        mesh=mesh,
        compiler_params=pltpu.CompilerParams(needs_layout_passes=False),
        out_type=[
            jax.ShapeDtypeStruct((NW, 16), jnp.float32),
            jax.ShapeDtypeStruct((NW, 16), jnp.float32),
        ],
        scratch_types=[
            pltpu.VMEM((B,), jnp.int32),
            pltpu.VMEM((CHUNK, D), jnp.float32),
            pltpu.VMEM((CHUNK, D), jnp.float32),
            pltpu.VMEM((CHUNK, D), jnp.float32),
            pltpu.VMEM((CHUNK, D), jnp.float32),
            pltpu.VMEM((B * CHUNK,), jnp.float32),
            pltpu.VMEM((B * CHUNK,), jnp.float32),
            pltpu.VMEM((16,), jnp.float32),
            pltpu.VMEM((16,), jnp.float32),
            pltpu.SemaphoreType.DMA,
            pltpu.SemaphoreType.DMA,
            pltpu.SemaphoreType.DMA,
        ],
    )
    def sc_kernel(len_hbm, oe_hbm, se_hbm, sw_hbm, sr_hbm,
                  wl_out, w_out,
                  len_v, a0_v, a1_v, c0_v, c1_v, w_v, r_v, awl_v, aw_v,
                  sem0, sem1, semwr):
        wid = lax.axis_index("s") * 2 + lax.axis_index("c")
        lane = lax.iota(jnp.int32, 16)

        for b in range(B):
            base = ((wid + 2 * b) & (NCHUNK - 1)) * CHUNK
            pltpu.async_copy(sw_hbm.at[b, pl.ds(base, CHUNK)],
                             w_v.at[pl.ds(b * CHUNK, CHUNK)], semwr)
            pltpu.async_copy(sr_hbm.at[b, pl.ds(base, CHUNK)],
                             r_v.at[pl.ds(b * CHUNK, CHUNK)], semwr)
        pltpu.sync_copy(len_hbm, len_v)
        for b in range(B):
            base = ((wid + 2 * b) & (NCHUNK - 1)) * CHUNK
            pltpu.make_async_copy(sw_hbm.at[b, pl.ds(base, CHUNK)],
                                  w_v.at[pl.ds(b * CHUNK, CHUNK)], semwr).wait()
            pltpu.make_async_copy(sr_hbm.at[b, pl.ds(base, CHUNK)],
                                  r_v.at[pl.ds(b * CHUNK, CHUNK)], semwr).wait()

        awl_v[...] = jnp.zeros((16,), jnp.float32)
        aw_v[...] = jnp.zeros((16,), jnp.float32)
        len_all = len_v[...]

        def binfo(b):
            base = ((wid + 2 * b) & (NCHUNK - 1)) * CHUNK
            len_b = jnp.max(jnp.where(lane == b, len_all, 0))
            return b * N + base, base, len_b

        def issue(b, a_buf, c_buf, sem):
            grow, base, len_b = binfo(b)

            @pl.when(base < len_b)
            def _():
                pltpu.async_copy(oe_hbm.at[pl.ds(grow, CHUNK)], a_buf, sem)
                pltpu.async_copy(se_hbm.at[pl.ds(grow, CHUNK)], c_buf, sem)

        def compute(b, a_buf, c_buf, sem):
            grow, base, len_b = binfo(b)

            @pl.when(base < len_b)
            def _():
                pltpu.make_async_copy(oe_hbm.at[pl.ds(grow, CHUNK)], a_buf, sem).wait()
                pltpu.make_async_copy(se_hbm.at[pl.ds(grow, CHUNK)], c_buf, sem).wait()

                def group_body(g, _):
                    row0 = g * 16
                    rows = row0 + lane
                    accs = [jnp.zeros((16,), jnp.float32) for _ in range(4)]
                    for d in range(D):
                        dvec = (lane + d) & (D - 1)
                        a = plsc.load_gather(a_buf, [rows, dvec])
                        c = plsc.load_gather(c_buf, [rows, dvec])
                        accs[d & 3] = accs[d & 3] + a * c
                    dots = (accs[0] + accs[1]) + (accs[2] + accs[3])
                    off = b * CHUNK + row0
                    t = r_v[pl.ds(off, 16)]
                    wv = w_v[pl.ds(off, 16)]
                    valid = (base + rows) < len_b
                    wv = jnp.where(valid, wv, 0.0)
                    awl_v[...] += _bce_weighted(dots, t, wv)
                    aw_v[...] += wv
                    return _

                lax.fori_loop(0, GROUPS, group_body, None)

        issue(0, a0_v, c0_v, sem0)

        def pipe_body(i, _):
            b0 = 2 * i
            issue(b0 + 1, a1_v, c1_v, sem1)
            compute(b0, a0_v, c0_v, sem0)

            @pl.when(i < B // 2 - 1)
            def _():
                issue(b0 + 2, a0_v, c0_v, sem0)

            compute(b0 + 1, a1_v, c1_v, sem1)
            return _

        lax.fori_loop(0, B // 2, pipe_body, None)

        pltpu.sync_copy(awl_v, wl_out.at[wid])
        pltpu.sync_copy(aw_v, w_out.at[wid])

    return sc_kernel(lengths,
                     output_embeddings.reshape(B * N, D),
                     supervision_embeddings.reshape(B * N, D),
                     supervision_weights, supervision_ratings)


def kernel(lengths, output_embeddings, supervision_ids, supervision_embeddings,
           supervision_weights, supervision_ratings):
    del supervision_ids
    wl, w = _sc_loss_parts(lengths, output_embeddings, supervision_embeddings,
                           supervision_weights, supervision_ratings)
    return jnp.sum(wl) / jnp.sum(w)
